# baseline (speedup 1.0000x reference)
import functools

import jax
import jax.numpy as jnp
from jax import lax
from jax.experimental import pallas as pl
from jax.experimental.pallas import tpu as pltpu
from jax.experimental.pallas import tpu_sc as plsc

B, T, D = 4, 8192, 1024
PRIOR_WEIGHT = 0.5
RUNTIME_WEIGHT = 0.5

T_BLK = 256
T_SC = 2048
T_TC = T - T_SC
N_WORKERS = 32
W_PER_B = N_WORKERS // B
MY_T = T_SC // W_PER_B
CHUNK = 16
N_CHUNKS = MY_T // CHUNK
NV = D // 16


def _tc_stage1_body(h_ref, s_ref, w_ref, rsum_ref, prior_ref):
    h = h_ref[...]
    s = s_ref[...]
    d = h - s
    rsum_ref[...] = jnp.sum(d * d, axis=2)
    hb = h.astype(jnp.bfloat16).astype(jnp.float32)
    wb = w_ref[0].astype(jnp.bfloat16).astype(jnp.float32)[None, None, :]
    prior_ref[...] = jnp.sum(hb * wb, axis=2)


def _round_bf16_i32(x):
    u = lax.bitcast_convert_type(x, jnp.int32)
    rounded = (u + 0x7FFF + ((u >> 16) & 1)) & ~0xFFFF
    return lax.bitcast_convert_type(rounded, jnp.float32)


_GATHER_DNUMS = lax.GatherDimensionNumbers(
    offset_dims=(), collapsed_slice_dims=(0,), start_index_map=(0,))


def _bcast_lane(x, u):
    idx = jnp.full((16, 1), u, jnp.int32)
    return lax.gather(x, idx, _GATHER_DNUMS, slice_sizes=(1,),
                      mode=lax.GatherScatterMode.PROMISE_IN_BOUNDS)


def _round_bf16_fast(x):
    u = lax.bitcast_convert_type(x, jnp.int32)
    rounded = (u + 0x8000) & ~0xFFFF
    return lax.bitcast_convert_type(rounded, jnp.float32)


def _round_bf16_pack(v0, v1):
    return _round_bf16_fast(v0), _round_bf16_fast(v1)


def _sc_stage1(h_hbm, s_hbm, w_hbm, rsum_out, prior_out,
               h_buf0, s_buf0, h_buf1, s_buf1, w_buf,
               out_r_buf, out_p_buf, sem_h0, sem_s0, sem_h1, sem_s1):
    wid = lax.axis_index("s") * 2 + lax.axis_index("c")
    b = wid // W_PER_B
    t0 = T_TC + (wid % W_PER_B) * MY_T

    pltpu.sync_copy(w_hbm.at[0], w_buf)
    lane = lax.iota(jnp.int32, 16)
    lane_d = lane * D

    def chunk_base(g):
        return (b * T + t0 + g * CHUNK) * D

    def start(g, h_buf, s_buf, sem_h, sem_s):
        pltpu.async_copy(h_hbm.at[pl.ds(chunk_base(g), CHUNK * D)],
                         h_buf, sem_h)
        pltpu.async_copy(s_hbm.at[pl.ds(chunk_base(g), CHUNK * D)],
                         s_buf, sem_s)

    UNROLL = 16

    def process(g, h_buf, s_buf, sem_h, sem_s):
        pltpu.make_async_copy(
            h_hbm.at[pl.ds(chunk_base(g), CHUNK * D)], h_buf, sem_h
        ).wait()
        pltpu.make_async_copy(
            s_hbm.at[pl.ds(chunk_base(g), CHUNK * D)], s_buf, sem_s
        ).wait()

        def do_d(jg, carry):
            acc = list(carry)
            wchunk = w_buf[pl.ds(jg * UNROLL, UNROLL)]
            idx0 = lane_d + jg * UNROLL
            for u in range(UNROLL):
                idx = idx0 + u
                hv = plsc.load_gather(h_buf, [idx])
                sv = plsc.load_gather(s_buf, [idx])
                wv = _bcast_lane(wchunk, u)
                dv = hv - sv
                k = u % 4
                acc[k] = acc[k] + dv * dv
                acc[4 + k] = acc[4 + k] + _round_bf16_i32(hv) * wv
            return tuple(acc)

        zero = jnp.zeros((16,), jnp.float32)
        acc = lax.fori_loop(0, D // UNROLL, do_d, (zero,) * 8)
        out_r_buf[pl.ds(g * CHUNK, CHUNK)] = acc[0] + acc[1] + acc[2] + acc[3]
        out_p_buf[pl.ds(g * CHUNK, CHUNK)] = acc[4] + acc[5] + acc[6] + acc[7]

        @pl.when(g + 2 < N_CHUNKS)
        def _prefetch():
            start(g + 2, h_buf, s_buf, sem_h, sem_s)

    start(0, h_buf0, s_buf0, sem_h0, sem_s0)
    start(1, h_buf1, s_buf1, sem_h1, sem_s1)

    def chunk_body(g):
        @pl.when(g % 2 == 0)
        def _even():
            process(g, h_buf0, s_buf0, sem_h0, sem_s0)

        @pl.when(g % 2 == 1)
        def _odd():
            process(g, h_buf1, s_buf1, sem_h1, sem_s1)

    pl.loop(0, N_CHUNKS)(chunk_body)
    pltpu.sync_copy(out_r_buf, rsum_out.at[b, pl.ds(t0 - T_TC, MY_T)])
    pltpu.sync_copy(out_p_buf, prior_out.at[b, pl.ds(t0 - T_TC, MY_T)])


def _stage2_body(rsum_tc_ref, rsum_sc_ref, prior_tc_ref, prior_sc_ref,
                 scores_ref, sem_ref, pscore_ref, rscore_ref, peak_ref):
    rsum = jnp.concatenate([rsum_tc_ref[...], rsum_sc_ref[...]], axis=1)
    praw = jnp.concatenate([prior_tc_ref[...], prior_sc_ref[...]], axis=1)
    rraw = jnp.sqrt(rsum)

    def standardize(x):
        mean = jnp.mean(x, axis=1, keepdims=True)
        var = jnp.mean((x - mean) * (x - mean), axis=1, keepdims=True)
        std = jnp.maximum(jnp.sqrt(var), 1e-6)
        return (x - mean) / std

    runtime_logits = standardize(rraw)
    prior_logits = standardize(praw)
    combined = PRIOR_WEIGHT * prior_logits + RUNTIME_WEIGHT * runtime_logits
    scores = jax.nn.sigmoid(combined)
    scores_ref[...] = scores
    sem_ref[...] = combined
    pscore_ref[...] = jax.nn.sigmoid(prior_logits)
    rscore_ref[...] = jax.nn.sigmoid(runtime_logits)
    left = jnp.concatenate([scores[:, :1], scores[:, :-1]], axis=1)
    right = jnp.concatenate([scores[:, 1:], scores[:, -1:]], axis=1)
    peak_ref[...] = ((scores >= left) & (scores >= right)).astype(jnp.int8)


def kernel(hidden, history, W, b):
    del b
    Wb = _round_bf16_i32(W)

    rsum_tc, prior_tc = pl.pallas_call(
        _tc_stage1_body,
        grid=(T_TC // T_BLK,),
        in_specs=[
            pl.BlockSpec((B, T_BLK, D), lambda j: (0, j, 0)),
            pl.BlockSpec((B, T_BLK, D), lambda j: (0, j, 0)),
            pl.BlockSpec((1, D), lambda j: (0, 0)),
        ],
        out_specs=[
            pl.BlockSpec((B, T_BLK), lambda j: (0, j)),
            pl.BlockSpec((B, T_BLK), lambda j: (0, j)),
        ],
        out_shape=[
            jax.ShapeDtypeStruct((B, T_TC), jnp.float32),
            jax.ShapeDtypeStruct((B, T_TC), jnp.float32),
        ],
    )(hidden, history, Wb)

    mesh = plsc.VectorSubcoreMesh(core_axis_name="c", subcore_axis_name="s")
    sc_call = functools.partial(
        pl.kernel, mesh=mesh,
        compiler_params=pltpu.CompilerParams(needs_layout_passes=False),
        out_type=[
            jax.ShapeDtypeStruct((B, T_SC), jnp.float32),
            jax.ShapeDtypeStruct((B, T_SC), jnp.float32),
        ],
        scratch_types=[
            pltpu.VMEM((CHUNK * D,), jnp.float32),
            pltpu.VMEM((CHUNK * D,), jnp.float32),
            pltpu.VMEM((CHUNK * D,), jnp.float32),
            pltpu.VMEM((CHUNK * D,), jnp.float32),
            pltpu.VMEM((D,), jnp.float32),
            pltpu.VMEM((MY_T,), jnp.float32),
            pltpu.VMEM((MY_T,), jnp.float32),
            pltpu.SemaphoreType.DMA,
            pltpu.SemaphoreType.DMA,
            pltpu.SemaphoreType.DMA,
            pltpu.SemaphoreType.DMA,
        ],
    )
    rsum_sc, prior_sc = sc_call(_sc_stage1)(
        hidden.reshape(-1), history.reshape(-1), Wb)

    scores, sem, pscore, rscore, peak_i8 = pl.pallas_call(
        _stage2_body,
        out_shape=[
            jax.ShapeDtypeStruct((B, T), jnp.float32),
            jax.ShapeDtypeStruct((B, T), jnp.float32),
            jax.ShapeDtypeStruct((B, T), jnp.float32),
            jax.ShapeDtypeStruct((B, T), jnp.float32),
            jax.ShapeDtypeStruct((B, T), jnp.int8),
        ],
    )(rsum_tc, rsum_sc, prior_tc, prior_sc)

    positions = jnp.arange(T, dtype=jnp.int32)
    starts = jnp.clip(positions - 1, 0, None)
    span_bounds = jnp.broadcast_to(
        jnp.stack((starts, positions), axis=-1)[None, :, :], (B, T, 2))

    return scores, span_bounds, sem, pscore, rscore, peak_i8.astype(jnp.bool_)


# speedup vs baseline: 4.7471x; 4.7471x over previous
import functools

import jax
import jax.numpy as jnp
from jax import lax
from jax.experimental import pallas as pl
from jax.experimental.pallas import tpu as pltpu
from jax.experimental.pallas import tpu_sc as plsc

B, T, D = 4, 8192, 1024
PRIOR_WEIGHT = 0.5
RUNTIME_WEIGHT = 0.5

T_BLK = 256
T_SC = 1280
T_TC = T - T_SC
N_WORKERS = 32
W_PER_B = N_WORKERS // B
MY_T = T_SC // W_PER_B
CHUNK = 16
N_CHUNKS = MY_T // CHUNK
NV = D // 16


def _tc_stage1_body(h_ref, s_ref, w_ref, rsum_ref, prior_ref):
    h = h_ref[...]
    s = s_ref[...]
    d = h - s
    rsum_ref[...] = jnp.sum(d * d, axis=2)
    hb = h.astype(jnp.bfloat16).astype(jnp.float32)
    wb = w_ref[0].astype(jnp.bfloat16).astype(jnp.float32)[None, None, :]
    prior_ref[...] = jnp.sum(hb * wb, axis=2)


def _round_bf16_i32(x):
    u = lax.bitcast_convert_type(x, jnp.int32)
    rounded = (u + 0x7FFF + ((u >> 16) & 1)) & ~0xFFFF
    return lax.bitcast_convert_type(rounded, jnp.float32)


_GATHER_DNUMS = lax.GatherDimensionNumbers(
    offset_dims=(), collapsed_slice_dims=(0,), start_index_map=(0,))


def _lane_sum(x):
    lane = lax.iota(jnp.int32, 16)
    for k in (1, 2, 4, 8):
        idx = (lane ^ k)[:, None]
        x = x + lax.gather(x, idx, _GATHER_DNUMS, slice_sizes=(1,),
                           mode=lax.GatherScatterMode.PROMISE_IN_BOUNDS)
    return x


def _round_bf16_fast(x):
    u = lax.bitcast_convert_type(x, jnp.int32)
    rounded = (u + 0x8000) & ~0xFFFF
    return lax.bitcast_convert_type(rounded, jnp.float32)


def _round_bf16_pack(v0, v1):
    return _round_bf16_fast(v0), _round_bf16_fast(v1)


def _sc_stage1(h_hbm, s_hbm, w_hbm, rsum_out, prior_out,
               h_buf0, s_buf0, h_buf1, s_buf1, w_buf,
               out_r_buf, out_p_buf, sem_h0, sem_s0, sem_h1, sem_s1):
    wid = lax.axis_index("s") * 2 + lax.axis_index("c")
    b = wid // W_PER_B
    t0 = T_TC + (wid % W_PER_B) * MY_T

    pltpu.sync_copy(w_hbm.at[0], w_buf)
    lane = lax.iota(jnp.int32, 16)

    def start(g, h_buf, s_buf, sem_h, sem_s):
        pltpu.async_copy(h_hbm.at[b, pl.ds(t0 + g * CHUNK, CHUNK), :],
                         h_buf, sem_h)
        pltpu.async_copy(s_hbm.at[b, pl.ds(t0 + g * CHUNK, CHUNK), :],
                         s_buf, sem_s)

    def process(g, h_buf, s_buf, sem_h, sem_s):
        pltpu.make_async_copy(
            h_hbm.at[b, pl.ds(t0 + g * CHUNK, CHUNK), :], h_buf, sem_h
        ).wait()
        pltpu.make_async_copy(
            s_hbm.at[b, pl.ds(t0 + g * CHUNK, CHUNK), :], s_buf, sem_s
        ).wait()

        def do_row(r, carry):
            out_r, out_p = carry
            acc_r = [jnp.zeros((16,), jnp.float32) for _ in range(4)]
            acc_p = [jnp.zeros((16,), jnp.float32) for _ in range(4)]
            for j2 in range(NV // 2):
                j = 2 * j2
                hv0 = h_buf[r, pl.ds(16 * j, 16)]
                hv1 = h_buf[r, pl.ds(16 * (j + 1), 16)]
                sv0 = s_buf[r, pl.ds(16 * j, 16)]
                sv1 = s_buf[r, pl.ds(16 * (j + 1), 16)]
                wv0 = w_buf[pl.ds(16 * j, 16)]
                wv1 = w_buf[pl.ds(16 * (j + 1), 16)]
                d0 = hv0 - sv0
                d1 = hv1 - sv1
                hb0, hb1 = _round_bf16_pack(hv0, hv1)
                k = j2 % 4
                acc_r[k] = acc_r[k] + d0 * d0
                acc_p[k] = acc_p[k] + hb0 * wv0
                k = (j2 + 2) % 4
                acc_r[k] = acc_r[k] + d1 * d1
                acc_p[k] = acc_p[k] + hb1 * wv1
            rs = _lane_sum(acc_r[0] + acc_r[1] + acc_r[2] + acc_r[3])
            ps = _lane_sum(acc_p[0] + acc_p[1] + acc_p[2] + acc_p[3])
            out_r = jnp.where(lane == r, rs, out_r)
            out_p = jnp.where(lane == r, ps, out_p)
            return out_r, out_p

        zero = jnp.zeros((16,), jnp.float32)
        out_r, out_p = lax.fori_loop(0, CHUNK, do_row, (zero, zero))
        out_r_buf[pl.ds(g * CHUNK, CHUNK)] = out_r
        out_p_buf[pl.ds(g * CHUNK, CHUNK)] = out_p

        @pl.when(g + 2 < N_CHUNKS)
        def _prefetch():
            start(g + 2, h_buf, s_buf, sem_h, sem_s)

    start(0, h_buf0, s_buf0, sem_h0, sem_s0)
    start(1, h_buf1, s_buf1, sem_h1, sem_s1)

    def chunk_body(g):
        @pl.when(g % 2 == 0)
        def _even():
            process(g, h_buf0, s_buf0, sem_h0, sem_s0)

        @pl.when(g % 2 == 1)
        def _odd():
            process(g, h_buf1, s_buf1, sem_h1, sem_s1)

    pl.loop(0, N_CHUNKS)(chunk_body)
    off = b * T_SC + (t0 - T_TC)
    pltpu.sync_copy(out_r_buf, rsum_out.at[pl.ds(off, MY_T)])
    pltpu.sync_copy(out_p_buf, prior_out.at[pl.ds(off, MY_T)])


def _stage2_body(rsum_tc_ref, rsum_sc_ref, prior_tc_ref, prior_sc_ref,
                 scores_ref, sem_ref, pscore_ref, rscore_ref, peak_ref):
    rsum = jnp.concatenate([rsum_tc_ref[...], rsum_sc_ref[...]], axis=1)
    praw = jnp.concatenate([prior_tc_ref[...], prior_sc_ref[...]], axis=1)
    rraw = jnp.sqrt(rsum)

    def standardize(x):
        mean = jnp.mean(x, axis=1, keepdims=True)
        var = jnp.mean((x - mean) * (x - mean), axis=1, keepdims=True)
        std = jnp.maximum(jnp.sqrt(var), 1e-6)
        return (x - mean) / std

    runtime_logits = standardize(rraw)
    prior_logits = standardize(praw)
    combined = PRIOR_WEIGHT * prior_logits + RUNTIME_WEIGHT * runtime_logits
    scores = jax.nn.sigmoid(combined)
    scores_ref[...] = scores
    sem_ref[...] = combined
    pscore_ref[...] = jax.nn.sigmoid(prior_logits)
    rscore_ref[...] = jax.nn.sigmoid(runtime_logits)
    left = jnp.concatenate([scores[:, :1], scores[:, :-1]], axis=1)
    right = jnp.concatenate([scores[:, 1:], scores[:, -1:]], axis=1)
    peak_ref[...] = ((scores >= left) & (scores >= right)).astype(jnp.int8)


def kernel(hidden, history, W, b):
    del b
    Wb = _round_bf16_i32(W)

    rsum_tc, prior_tc = pl.pallas_call(
        _tc_stage1_body,
        grid=(T_TC // T_BLK,),
        in_specs=[
            pl.BlockSpec((B, T_BLK, D), lambda j: (0, j, 0)),
            pl.BlockSpec((B, T_BLK, D), lambda j: (0, j, 0)),
            pl.BlockSpec((1, D), lambda j: (0, 0)),
        ],
        out_specs=[
            pl.BlockSpec((B, T_BLK), lambda j: (0, j)),
            pl.BlockSpec((B, T_BLK), lambda j: (0, j)),
        ],
        out_shape=[
            jax.ShapeDtypeStruct((B, T_TC), jnp.float32),
            jax.ShapeDtypeStruct((B, T_TC), jnp.float32),
        ],
    )(hidden, history, Wb)

    mesh = plsc.VectorSubcoreMesh(core_axis_name="c", subcore_axis_name="s")
    sc_call = functools.partial(
        pl.kernel, mesh=mesh,
        out_type=[
            jax.ShapeDtypeStruct((B * T_SC,), jnp.float32),
            jax.ShapeDtypeStruct((B * T_SC,), jnp.float32),
        ],
        scratch_types=[
            pltpu.VMEM((CHUNK, D), jnp.float32),
            pltpu.VMEM((CHUNK, D), jnp.float32),
            pltpu.VMEM((CHUNK, D), jnp.float32),
            pltpu.VMEM((CHUNK, D), jnp.float32),
            pltpu.VMEM((D,), jnp.float32),
            pltpu.VMEM((MY_T,), jnp.float32),
            pltpu.VMEM((MY_T,), jnp.float32),
            pltpu.SemaphoreType.DMA,
            pltpu.SemaphoreType.DMA,
            pltpu.SemaphoreType.DMA,
            pltpu.SemaphoreType.DMA,
        ],
    )
    rsum_sc, prior_sc = sc_call(_sc_stage1)(hidden, history, Wb)
    rsum_sc = rsum_sc.reshape(B, T_SC)
    prior_sc = prior_sc.reshape(B, T_SC)

    scores, sem, pscore, rscore, peak_i8 = pl.pallas_call(
        _stage2_body,
        out_shape=[
            jax.ShapeDtypeStruct((B, T), jnp.float32),
            jax.ShapeDtypeStruct((B, T), jnp.float32),
            jax.ShapeDtypeStruct((B, T), jnp.float32),
            jax.ShapeDtypeStruct((B, T), jnp.float32),
            jax.ShapeDtypeStruct((B, T), jnp.int8),
        ],
    )(rsum_tc, rsum_sc, prior_tc, prior_sc)

    positions = jnp.arange(T, dtype=jnp.int32)
    starts = jnp.clip(positions - 1, 0, None)
    span_bounds = jnp.broadcast_to(
        jnp.stack((starts, positions), axis=-1)[None, :, :], (B, T, 2))

    return scores, span_bounds, sem, pscore, rscore, peak_i8.astype(jnp.bool_)
